# baseline (speedup 1.0000x reference)
import functools
import math

import jax
import jax.numpy as jnp
from jax import lax
from jax.experimental import pallas as pl
from jax.experimental.pallas import tpu as pltpu
from jax.experimental.pallas import tpu_sc as plsc

VOCAB = 100000
D_MODEL = 1024
BATCH = 4
SEQ_LEN = 4096
N_ROWS = BATCH * SEQ_LEN
SCALE = math.sqrt(D_MODEL)

_info = plsc.get_sparse_core_info()
NUM_CORES = _info.num_cores
NUM_SUBCORES = _info.num_subcores
LANES = _info.num_lanes
NW = NUM_CORES * NUM_SUBCORES
S_PER_W = SEQ_LEN // NW
K = 16
N_SCHUNK = S_PER_W // K
N_UNITS = N_SCHUNK * BATCH
G = 8
VECS_PER_ROW = D_MODEL // LANES
IDX_ROWS_PER_B = SEQ_LEN // K


def _emb_body(x_ref, tok_ref, pos_ref, out_ref,
              idx2d, rows0, rows1, posA, posB,
              semg0, semg1, sempA, sempB, semw0, semw1):
    wid = lax.axis_index("s") * NUM_CORES + lax.axis_index("c")
    s0 = wid * S_PER_W

    for b in range(BATCH):
        pltpu.sync_copy(
            x_ref.at[pl.ds(b * IDX_ROWS_PER_B + wid * N_SCHUNK, N_SCHUNK)],
            idx2d.at[pl.ds(b * N_SCHUNK, N_SCHUNK)])

    rows = (rows0, rows1)
    semg = (semg0, semg1)
    semw = (semw0, semw1)
    poss = (posA, posB)
    semp = (sempA, sempB)

    def unit_row0(u):
        sc = u // BATCH
        b = u % BATCH
        return b * SEQ_LEN + s0 + sc * K

    def gather_desc(u, buf):
        urow = (u % BATCH) * N_SCHUNK + u // BATCH
        return pltpu.make_async_copy(tok_ref.at[idx2d.at[urow]],
                                     rows[buf], semg[buf])

    def pos_desc(sc, slot):
        return pltpu.make_async_copy(pos_ref.at[pl.ds(s0 + sc * K, K)],
                                     poss[slot], semp[slot])

    def wb_desc(u, buf):
        return pltpu.make_async_copy(rows[buf],
                                     out_ref.at[pl.ds(unit_row0(u), K)],
                                     semw[buf])

    def compute(buf, slot):
        rows_b, pos_b = rows[buf], poss[slot]

        def row_body(r, carry):
            for v in range(VECS_PER_ROW):
                sl = pl.ds(v * LANES, LANES)
                rows_b[r, sl] = rows_b[r, sl] * SCALE + pos_b[r, sl]
            return carry

        lax.fori_loop(0, K, row_body, 0, unroll=1)

    def group_body(sp, carry):
        for j in range(G):
            u = G * sp + j
            buf = j % 2
            if j >= 2:
                wb_desc(u - 2, buf).wait()
            else:
                @pl.when(sp >= 1)
                def _():
                    wb_desc(u - 2, buf).wait()
            gather_desc(u, buf).start()
            if j % BATCH == 0:
                slot_new = j // BATCH
                pos_desc(2 * sp + j // BATCH, slot_new).start()
            ob = 1 - buf
            if j == 0:
                @pl.when(sp >= 1)
                def _():
                    gather_desc(u - 1, ob).wait()
                    compute(ob, 1)
                    wb_desc(u - 1, ob).start()
            else:
                gather_desc(u - 1, ob).wait()
                if j in (1, 1 + BATCH):
                    pos_desc(0, (j - 1) // BATCH).wait()
                compute(ob, (j - 1) // BATCH)
                wb_desc(u - 1, ob).start()
        return carry

    lax.fori_loop(0, N_UNITS // G, group_body, 0, unroll=1)

    last = N_UNITS - 1
    gather_desc(last, 1).wait()
    compute(1, 1)
    wb_desc(last, 1).start()
    wb_desc(last - 1, 0).wait()
    wb_desc(last, 1).wait()


@jax.jit
def _emb_call(x2d, token_table, pos_table):
    mesh = plsc.VectorSubcoreMesh(core_axis_name="c", subcore_axis_name="s")
    f = functools.partial(
        pl.kernel,
        out_type=jax.ShapeDtypeStruct((N_ROWS, D_MODEL), jnp.float32),
        mesh=mesh,
        scratch_types=[
            pltpu.VMEM((N_UNITS, K), jnp.int32),
            pltpu.VMEM((K, D_MODEL), jnp.float32),
            pltpu.VMEM((K, D_MODEL), jnp.float32),
            pltpu.VMEM((K, D_MODEL), jnp.float32),
            pltpu.VMEM((K, D_MODEL), jnp.float32),
            pltpu.SemaphoreType.DMA,
            pltpu.SemaphoreType.DMA,
            pltpu.SemaphoreType.DMA,
            pltpu.SemaphoreType.DMA,
            pltpu.SemaphoreType.DMA,
            pltpu.SemaphoreType.DMA,
        ],
    )(_emb_body)
    return f(x2d, token_table, pos_table)


def kernel(x, token_table, pos_table):
    x2d = x.reshape(N_ROWS // K, K).astype(jnp.int32)
    out = _emb_call(x2d, token_table, pos_table)
    return out.reshape(BATCH, SEQ_LEN, D_MODEL)


# speedup vs baseline: 1.3544x; 1.3544x over previous
import functools
import math

import jax
import jax.numpy as jnp
from jax import lax
from jax.experimental import pallas as pl
from jax.experimental.pallas import tpu as pltpu
from jax.experimental.pallas import tpu_sc as plsc

VOCAB = 100000
D_MODEL = 1024
BATCH = 4
SEQ_LEN = 4096
N_ROWS = BATCH * SEQ_LEN
SCALE = math.sqrt(D_MODEL)

_info = plsc.get_sparse_core_info()
NUM_CORES = _info.num_cores
NUM_SUBCORES = _info.num_subcores
LANES = _info.num_lanes
NW = NUM_CORES * NUM_SUBCORES
S_PER_W = SEQ_LEN // NW
K = 8
N_GROUPS = S_PER_W // K
VECS_PER_ROW = D_MODEL // LANES
IDX_ROWS_PER_B = SEQ_LEN // K


def _emb_body(x_ref, tok_ref, pos_ref, out_ref,
              idx2d, rowsbuf, posbuf,
              semg0, semg1, semp0, semp1, semw0, semw1):
    wid = lax.axis_index("s") * NUM_CORES + lax.axis_index("c")
    s0 = wid * S_PER_W

    for b in range(BATCH):
        pltpu.sync_copy(
            x_ref.at[pl.ds(b * IDX_ROWS_PER_B + wid * N_GROUPS, N_GROUPS)],
            idx2d.at[pl.ds(b * N_GROUPS, N_GROUPS)])

    semg = (semg0, semg1)
    semp = (semp0, semp1)
    semw = (semw0, semw1)

    def gather_desc(g, par, b):
        urow = b * N_GROUPS + g
        return pltpu.make_async_copy(tok_ref.at[idx2d.at[urow]],
                                     rowsbuf.at[par, b], semg[par])

    def pos_desc(g, par):
        return pltpu.make_async_copy(pos_ref.at[pl.ds(s0 + g * K, K)],
                                     posbuf.at[par], semp[par])

    def wb_desc(g, par, b):
        row0 = b * SEQ_LEN + s0 + g * K
        return pltpu.make_async_copy(rowsbuf.at[par, b],
                                     out_ref.at[pl.ds(row0, K)], semw[par])

    def compute(par):
        def row_body(r, carry):
            for v in range(VECS_PER_ROW):
                sl = pl.ds(v * LANES, LANES)
                pv = posbuf[par, r, sl]
                for b in range(BATCH):
                    rowsbuf[par, b, r, sl] = rowsbuf[par, b, r, sl] * SCALE + pv
            return carry

        lax.fori_loop(0, K, row_body, 0, unroll=1)

    def start_group(g, par):
        pos_desc(g, par).start()
        for b in range(BATCH):
            gather_desc(g, par, b).start()

    def finish_group(g, par):
        pos_desc(g, par).wait()
        for b in range(BATCH):
            gather_desc(g, par, b).wait()
        compute(par)
        for b in range(BATCH):
            wb_desc(g, par, b).start()

    def drain_group(g, par):
        for b in range(BATCH):
            wb_desc(g, par, b).wait()

    def pair_body(gp, carry):
        for par in (0, 1):
            g = 2 * gp + par
            @pl.when(gp >= 1)
            def _():
                drain_group(g - 2, par)
            start_group(g, par)
            if par == 0:
                @pl.when(gp >= 1)
                def _():
                    finish_group(g - 1, 1)
            else:
                finish_group(g - 1, 0)
        return carry

    lax.fori_loop(0, N_GROUPS // 2, pair_body, 0, unroll=1)

    last = N_GROUPS - 1
    finish_group(last, 1)
    drain_group(last - 1, 0)
    drain_group(last, 1)


@jax.jit
def _emb_call(x2d, token_table, pos_table):
    mesh = plsc.VectorSubcoreMesh(core_axis_name="c", subcore_axis_name="s")
    f = functools.partial(
        pl.kernel,
        out_type=jax.ShapeDtypeStruct((N_ROWS, D_MODEL), jnp.float32),
        mesh=mesh,
        scratch_types=[
            pltpu.VMEM((BATCH * N_GROUPS, K), jnp.int32),
            pltpu.VMEM((2, BATCH, K, D_MODEL), jnp.float32),
            pltpu.VMEM((2, K, D_MODEL), jnp.float32),
            pltpu.SemaphoreType.DMA,
            pltpu.SemaphoreType.DMA,
            pltpu.SemaphoreType.DMA,
            pltpu.SemaphoreType.DMA,
            pltpu.SemaphoreType.DMA,
            pltpu.SemaphoreType.DMA,
        ],
    )(_emb_body)
    return f(x2d, token_table, pos_table)


def kernel(x, token_table, pos_table):
    x2d = x.reshape(N_ROWS // K, K).astype(jnp.int32)
    out = _emb_call(x2d, token_table, pos_table)
    return out.reshape(BATCH, SEQ_LEN, D_MODEL)


# speedup vs baseline: 1.3604x; 1.0044x over previous
import functools
import math

import jax
import jax.numpy as jnp
from jax import lax
from jax.experimental import pallas as pl
from jax.experimental.pallas import tpu as pltpu
from jax.experimental.pallas import tpu_sc as plsc

VOCAB = 100000
D_MODEL = 1024
BATCH = 4
SEQ_LEN = 4096
N_ROWS = BATCH * SEQ_LEN
SCALE = math.sqrt(D_MODEL)

_info = plsc.get_sparse_core_info()
NUM_CORES = _info.num_cores
NUM_SUBCORES = _info.num_subcores
LANES = _info.num_lanes
NW = NUM_CORES * NUM_SUBCORES
S_PER_W = SEQ_LEN // NW
K = 8
GR = BATCH * K
N_GROUPS = S_PER_W // K
SETS = 3
VECS_PER_ROW = D_MODEL // LANES
IDX_ROWS_PER_B = SEQ_LEN // K


def _emb_body(x_ref, tok_ref, pos_ref, out_ref,
              idx3d, rowsbuf, posbuf,
              semg0, semg1, semg2, semp0, semp1, semp2,
              semw0, semw1, semw2):
    wid = lax.axis_index("s") * NUM_CORES + lax.axis_index("c")
    s0 = wid * S_PER_W

    pltpu.sync_copy(x_ref.at[pl.ds(wid * N_GROUPS, N_GROUPS)], idx3d)

    semg = (semg0, semg1, semg2)
    semp = (semp0, semp1, semp2)
    semw = (semw0, semw1, semw2)

    def gather_desc(g, par):
        return pltpu.make_async_copy(tok_ref.at[idx3d.at[g]],
                                     rowsbuf.at[par], semg[par])

    def pos_desc(g, par):
        return pltpu.make_async_copy(pos_ref.at[pl.ds(s0 + g * K, K)],
                                     posbuf.at[par], semp[par])

    def wb_desc(g, par, b):
        row0 = b * SEQ_LEN + s0 + g * K
        return pltpu.make_async_copy(rowsbuf.at[par, pl.ds(b * K, K)],
                                     out_ref.at[pl.ds(row0, K)], semw[par])

    def compute(par):
        def row_body(r, carry):
            for v in range(VECS_PER_ROW):
                sl = pl.ds(v * LANES, LANES)
                pv = posbuf[par, r, sl]
                for b in range(BATCH):
                    row = b * K + r
                    rowsbuf[par, row, sl] = rowsbuf[par, row, sl] * SCALE + pv
            return carry

        lax.fori_loop(0, K, row_body, 0, unroll=1)

    def start_group(g, par):
        pos_desc(g, par).start()
        gather_desc(g, par).start()

    def finish_group(g, par):
        pos_desc(g, par).wait()
        gather_desc(g, par).wait()
        compute(par)
        for b in range(BATCH):
            wb_desc(g, par, b).start()

    def drain_group(g, par):
        for b in range(BATCH):
            wb_desc(g, par, b).wait()

    def round_body(rp, carry):
        for j in range(SETS):
            g = SETS * rp + j
            @pl.when(rp >= 1)
            def _():
                drain_group(g - SETS, j)
            start_group(g, j)
            pj = (j - 1) % SETS
            if j == 0:
                @pl.when(rp >= 1)
                def _():
                    finish_group(g - 1, pj)
            else:
                finish_group(g - 1, pj)
        return carry

    n_loop = (N_GROUPS // SETS) * SETS
    lax.fori_loop(0, N_GROUPS // SETS, round_body, 0, unroll=1)

    for g in range(n_loop, N_GROUPS):
        par = g % SETS
        drain_group(g - SETS, par)
        start_group(g, par)
        finish_group(g - 1, (g - 1) % SETS)
    finish_group(N_GROUPS - 1, (N_GROUPS - 1) % SETS)
    for g in range(N_GROUPS - SETS, N_GROUPS):
        drain_group(g, g % SETS)


@jax.jit
def _emb_call(x2d, token_table, pos_table):
    mesh = plsc.VectorSubcoreMesh(core_axis_name="c", subcore_axis_name="s")
    f = functools.partial(
        pl.kernel,
        out_type=jax.ShapeDtypeStruct((N_ROWS, D_MODEL), jnp.float32),
        mesh=mesh,
        scratch_types=[
            pltpu.VMEM((N_GROUPS, GR), jnp.int32),
            pltpu.VMEM((SETS, GR, D_MODEL), jnp.float32),
            pltpu.VMEM((SETS, K, D_MODEL), jnp.float32),
            pltpu.SemaphoreType.DMA,
            pltpu.SemaphoreType.DMA,
            pltpu.SemaphoreType.DMA,
            pltpu.SemaphoreType.DMA,
            pltpu.SemaphoreType.DMA,
            pltpu.SemaphoreType.DMA,
            pltpu.SemaphoreType.DMA,
            pltpu.SemaphoreType.DMA,
            pltpu.SemaphoreType.DMA,
        ],
    )(_emb_body)
    return f(x2d, token_table, pos_table)


def kernel(x, token_table, pos_table):
    xp = (x.astype(jnp.int32)
          .reshape(BATCH, NW, N_GROUPS, K)
          .transpose(1, 2, 0, 3)
          .reshape(NW * N_GROUPS, GR))
    out = _emb_call(xp, token_table, pos_table)
    return out.reshape(BATCH, SEQ_LEN, D_MODEL)
